# Initial kernel scaffold; baseline (speedup 1.0000x reference)
import jax
import jax.numpy as jnp
from jax.experimental import pallas as pl


def kernel(indices, codebook):
    raise NotImplementedError("write your pallas kernel here")


# speedup vs baseline: 3.3630x; 3.3630x over previous
import functools

import jax
import jax.numpy as jnp
from jax import lax
from jax.experimental import pallas as pl
from jax.experimental.pallas import tpu as pltpu
from jax.experimental.pallas import tpu_sc as plsc

_NC = 2
_NS = 16
_NW = _NC * _NS
_CHUNK = 128


def _make_gather(n, d):
    per_w = n // _NW
    n_chunks = per_w // _CHUNK
    mesh = plsc.VectorSubcoreMesh(core_axis_name="c", subcore_axis_name="s")

    @functools.partial(
        pl.kernel,
        mesh=mesh,
        out_type=jax.ShapeDtypeStruct((n, d), jnp.float32),
        scratch_types=[
            pltpu.VMEM((per_w,), jnp.int32),
            pltpu.VMEM((_CHUNK, d), jnp.float32),
            pltpu.SemaphoreType.DMA,
        ],
        compiler_params=pltpu.CompilerParams(use_tc_tiling_on_sc=False),
    )
    def k(idx_hbm, table_hbm, out_hbm, idx_v, rows_v, sem):
        wid = lax.axis_index("s") * _NC + lax.axis_index("c")
        base = wid * per_w
        pltpu.sync_copy(idx_hbm.at[pl.ds(base, per_w)], idx_v)

        def body(g, carry):
            off = g * _CHUNK
            pltpu.async_copy(
                table_hbm.at[idx_v.at[pl.ds(off, _CHUNK)]], rows_v, sem
            ).wait()
            pltpu.sync_copy(rows_v, out_hbm.at[pl.ds(base + off, _CHUNK)])
            return carry

        lax.fori_loop(0, n_chunks, body, 0)

    return k


def kernel(indices, codebook):
    b, t = indices.shape
    d = codebook.shape[1]
    n = b * t
    flat_idx = indices.reshape(n).astype(jnp.int32)
    out = _make_gather(n, d)(flat_idx, codebook)
    return out.reshape(b, t, d)


# speedup vs baseline: 3.8869x; 1.1558x over previous
import functools

import jax
import jax.numpy as jnp
from jax import lax
from jax.experimental import pallas as pl
from jax.experimental.pallas import tpu as pltpu
from jax.experimental.pallas import tpu_sc as plsc

_NC = 2
_NS = 16
_NW = _NC * _NS
_CHUNK = 128


_NB = 4


def _make_gather(n, d):
    per_w = n // _NW
    n_chunks = per_w // _CHUNK
    mesh = plsc.VectorSubcoreMesh(core_axis_name="c", subcore_axis_name="s")

    @functools.partial(
        pl.kernel,
        mesh=mesh,
        out_type=jax.ShapeDtypeStruct((n, d), jnp.float32),
        scratch_types=[
            pltpu.VMEM((per_w,), jnp.int32),
            [pltpu.VMEM((_CHUNK, d), jnp.float32) for _ in range(_NB)],
            [pltpu.SemaphoreType.DMA for _ in range(_NB)],
            [pltpu.SemaphoreType.DMA for _ in range(_NB)],
        ],
        compiler_params=pltpu.CompilerParams(use_tc_tiling_on_sc=False),
    )
    def k(idx_hbm, table_hbm, out_hbm, idx_v, bufs, gsems, wsems):
        wid = lax.axis_index("s") * _NC + lax.axis_index("c")
        base = wid * per_w
        pltpu.sync_copy(idx_hbm.at[pl.ds(base, per_w)], idx_v)

        def gather(g, b):
            return pltpu.make_async_copy(
                table_hbm.at[idx_v.at[pl.ds(g * _CHUNK, _CHUNK)]],
                bufs[b],
                gsems[b],
            )

        def write(g, b):
            return pltpu.make_async_copy(
                bufs[b],
                out_hbm.at[pl.ds(base + g * _CHUNK, _CHUNK)],
                wsems[b],
            )

        for b in range(_NB):
            gather(b, b).start()
        for g in range(n_chunks):
            b = g % _NB
            gather(g, b).wait()
            write(g, b).start()
            nxt = g + _NB
            if nxt < n_chunks:
                write(g, b).wait()
                gather(nxt, b).start()
        for g in range(max(0, n_chunks - _NB), n_chunks):
            write(g, g % _NB).wait()

    return k


def kernel(indices, codebook):
    b, t = indices.shape
    d = codebook.shape[1]
    n = b * t
    flat_idx = indices.reshape(n).astype(jnp.int32)
    out = _make_gather(n, d)(flat_idx, codebook)
    return out.reshape(b, t, d)
